# baseline (speedup 1.0000x reference)
import functools

import jax
import jax.numpy as jnp
from jax.experimental import pallas as pl

D = 64
NHEAD = 8
HD = D // NHEAD
KTOP = 32
N = 1024

_HIGHEST = jax.lax.Precision.DEFAULT


def _rowsum(m):
    return jnp.sum(m, axis=1, keepdims=True)


def _layernorm(y, g, b):
    mu = jnp.mean(y, axis=1, keepdims=True)
    c = y - mu
    var = jnp.mean(c * c, axis=1, keepdims=True)
    return c / jnp.sqrt(var + 1e-5) * g + b


NSAMP = 2


def _gcn_body(x_ref, Uw_ref, Ub_ref, Vw_ref, Vb_ref, bng_ref, bnb_ref,
              bnm_ref, bnv_ref, ipw_ref, ipb_ref, opw_ref, opb_ref,
              l1w_ref, l1b_ref, l2w_ref, l2b_ref, ln1g_ref, ln1b_ref,
              ln2g_ref, ln2b_ref, out_ref):
    f32 = jnp.float32
    bf16 = jnp.bfloat16
    S = NSAMP
    xs = [x_ref[i] for i in range(S)]

    sis = []
    for xb in xs:
        nrm = jnp.sqrt(_rowsum(xb * xb))
        sn = xb / jnp.maximum(nrm, 1e-12)
        sis.append(jax.lax.dot_general(sn, sn, (((1,), (1,)), ((), ())),
                                       preferred_element_type=f32,
                                       precision=_HIGHEST))

    keys = []
    for si in sis:
        siz = jnp.where(si == 0.0, 0.0, si)
        bits = jax.lax.bitcast_convert_type(siz, jnp.int32)
        keys.append(bits ^ ((bits >> 31) & jnp.int32(0x7FFFFFFF)))

    lo0 = jnp.full((N, 1), jnp.iinfo(jnp.int32).min, jnp.int32)
    hi0 = jnp.full((N, 1), jnp.iinfo(jnp.int32).max, jnp.int32)
    deg0 = jnp.full((N, 1), N, jnp.int32)

    ones_col = jnp.ones((N, 1), jnp.bfloat16)

    def bs_step(_, carry):
        out = []
        for s in range(S):
            lo, hi, deg = carry[3 * s:3 * s + 3]
            mid = (lo >> 1) + (hi >> 1) + (lo & hi & 1)
            mask = (keys[s] >= mid).astype(jnp.bfloat16)
            cnt = jax.lax.dot_general(mask, ones_col,
                                      (((1,), (0,)), ((), ())),
                                      preferred_element_type=jnp.float32)
            ge = cnt >= KTOP
            out += [jnp.where(ge, mid, lo), jnp.where(ge, hi, mid),
                    jnp.where(ge, cnt, deg)]
        return tuple(out)

    deg0f = jnp.full((N, 1), float(N), jnp.float32)
    fin = jax.lax.fori_loop(0, 32, bs_step, (lo0, hi0, deg0f) * S)

    x1s = []
    for s in range(S):
        lo, deg = fin[3 * s], fin[3 * s + 2]
        xb = xs[s]
        adj = (keys[s] >= lo).astype(f32)
        dinv = jax.lax.rsqrt(deg.astype(f32))
        vx = jax.lax.dot_general(xb, Vw_ref[...], (((1,), (1,)), ((), ())),
                                 preferred_element_type=f32,
                                 precision=_HIGHEST) + Vb_ref[...]
        agg = jax.lax.dot_general(adj, vx * dinv, (((1,), (0,)), ((), ())),
                                  preferred_element_type=f32,
                                  precision=_HIGHEST) * dinv
        ux = jax.lax.dot_general(xb, Uw_ref[...], (((1,), (1,)), ((), ())),
                                 preferred_element_type=f32,
                                 precision=_HIGHEST) + Ub_ref[...]
        res = agg + ux
        res = (res - bnm_ref[...]) / jnp.sqrt(bnv_ref[...] + 1e-5) \
            * bng_ref[...] + bnb_ref[...]
        x1s.append(jnp.maximum(xb + res, 0.0))

    scale = 1.0 / (HD ** 0.5)
    for s in range(S):
        x1 = x1s[s]
        qkv = jax.lax.dot_general(x1, ipw_ref[...], (((1,), (1,)), ((), ())),
                                  preferred_element_type=f32,
                                  precision=_HIGHEST) + ipb_ref[...]
        heads = []
        for h in range(NHEAD):
            qh = (qkv[:, h * HD:(h + 1) * HD] * scale).astype(bf16)
            kh = qkv[:, D + h * HD:D + (h + 1) * HD].astype(bf16)
            vh = qkv[:, 2 * D + h * HD:2 * D + (h + 1) * HD].astype(bf16)
            sc = jax.lax.dot_general(qh, kh, (((1,), (1,)), ((), ())),
                                     preferred_element_type=f32)
            m = jnp.max(sc, axis=1, keepdims=True)
            e = jnp.exp(sc - m)
            oh = jax.lax.dot_general(e.astype(bf16), vh,
                                     (((1,), (0,)), ((), ())),
                                     preferred_element_type=f32)
            heads.append(oh / _rowsum(e))
        o = jnp.concatenate(heads, axis=1)
        sa = jax.lax.dot_general(o, opw_ref[...], (((1,), (1,)), ((), ())),
                                 preferred_element_type=f32,
                                 precision=_HIGHEST) + opb_ref[...]

        x2 = _layernorm(x1 + sa, ln1g_ref[...], ln1b_ref[...])

        h1 = jnp.maximum(
            jax.lax.dot_general(x2, l1w_ref[...], (((1,), (1,)), ((), ())),
                                preferred_element_type=f32,
                                precision=_HIGHEST) + l1b_ref[...], 0.0)
        ff = jax.lax.dot_general(h1, l2w_ref[...], (((1,), (1,)), ((), ())),
                                 preferred_element_type=f32,
                                 precision=_HIGHEST) + l2b_ref[...]
        out_ref[s] = _layernorm(x2 + ff, ln2g_ref[...], ln2b_ref[...])


def _full(shape):
    return pl.BlockSpec(shape, lambda b: tuple(0 for _ in shape))


def _make_call(interpret=False):
    in_specs = [
        pl.BlockSpec((NSAMP, N, D), lambda b: (b, 0, 0)),
        _full((D, D)), _full((1, D)),
        _full((D, D)), _full((1, D)),
        _full((1, D)), _full((1, D)), _full((1, D)), _full((1, D)),
        _full((3 * D, D)), _full((1, 3 * D)),
        _full((D, D)), _full((1, D)),
        _full((D, D)), _full((1, D)),
        _full((D, D)), _full((1, D)),
        _full((1, D)), _full((1, D)),
        _full((1, D)), _full((1, D)),
    ]
    return pl.pallas_call(
        _gcn_body,
        grid=(8 // NSAMP,),
        in_specs=in_specs,
        out_specs=pl.BlockSpec((NSAMP, N, D), lambda b: (b, 0, 0)),
        out_shape=jax.ShapeDtypeStruct((8, N, D), jnp.float32),
        interpret=interpret,
    )


@jax.jit
def kernel(x, Uw, Ub, Vw, Vb, bn_gamma, bn_beta, bn_mean, bn_var,
           in_proj_w, in_proj_b, out_proj_w, out_proj_b,
           l1_w, l1_b, l2_w, l2_b, ln1_g, ln1_b, ln2_g, ln2_b):
    r = lambda v: v.reshape(1, -1)
    return _make_call()(
        x, Uw, r(Ub), Vw, r(Vb), r(bn_gamma), r(bn_beta), r(bn_mean),
        r(bn_var), in_proj_w, r(in_proj_b), out_proj_w, r(out_proj_b),
        l1_w, r(l1_b), l2_w, r(l2_b), r(ln1_g), r(ln1_b), r(ln2_g),
        r(ln2_b))


# speedup vs baseline: 1.0960x; 1.0960x over previous
import functools

import jax
import jax.numpy as jnp
from jax.experimental import pallas as pl

D = 64
NHEAD = 8
HD = D // NHEAD
KTOP = 32
N = 1024

_HIGHEST = jax.lax.Precision.DEFAULT


def _rowsum(m):
    return jnp.sum(m, axis=1, keepdims=True)


def _layernorm(y, g, b):
    mu = jnp.mean(y, axis=1, keepdims=True)
    c = y - mu
    var = jnp.mean(c * c, axis=1, keepdims=True)
    return c / jnp.sqrt(var + 1e-5) * g + b


NSAMP = 1


def _gcn_body(x_ref, Uw_ref, Ub_ref, Vw_ref, Vb_ref, bng_ref, bnb_ref,
              bnm_ref, bnv_ref, ipw_ref, ipb_ref, opw_ref, opb_ref,
              l1w_ref, l1b_ref, l2w_ref, l2b_ref, ln1g_ref, ln1b_ref,
              ln2g_ref, ln2b_ref, out_ref):
    f32 = jnp.float32
    bf16 = jnp.bfloat16
    S = NSAMP
    xs = [x_ref[i] for i in range(S)]

    sis = []
    for xb in xs:
        nrm = jnp.sqrt(_rowsum(xb * xb))
        sn = xb / jnp.maximum(nrm, 1e-12)
        sis.append(jax.lax.dot_general(sn, sn, (((1,), (1,)), ((), ())),
                                       preferred_element_type=f32,
                                       precision=_HIGHEST))

    keys = []
    for si in sis:
        siz = jnp.where(si == 0.0, 0.0, si)
        bits = jax.lax.bitcast_convert_type(siz, jnp.int32)
        keys.append(bits ^ ((bits >> 31) & jnp.int32(0x7FFFFFFF)))

    lo0 = jnp.full((N, 1), jnp.iinfo(jnp.int32).min, jnp.int32)
    hi0 = jnp.full((N, 1), jnp.iinfo(jnp.int32).max, jnp.int32)
    deg0 = jnp.full((N, 1), N, jnp.int32)

    ones_col = jnp.ones((N, 1), jnp.bfloat16)

    def bs_step(_, carry):
        out = []
        for s in range(S):
            lo, hi, deg = carry[3 * s:3 * s + 3]
            mid = (lo >> 1) + (hi >> 1) + (lo & hi & 1)
            mask = (keys[s] >= mid).astype(jnp.bfloat16)
            cnt = jax.lax.dot_general(mask, ones_col,
                                      (((1,), (0,)), ((), ())),
                                      preferred_element_type=jnp.float32)
            ge = cnt >= KTOP
            out += [jnp.where(ge, mid, lo), jnp.where(ge, hi, mid),
                    jnp.where(ge, cnt, deg)]
        return tuple(out)

    deg0f = jnp.full((N, 1), float(N), jnp.float32)
    fin = jax.lax.fori_loop(0, 32, bs_step, (lo0, hi0, deg0f) * S)

    x1s = []
    for s in range(S):
        lo, deg = fin[3 * s], fin[3 * s + 2]
        xb = xs[s]
        adj = (keys[s] >= lo).astype(f32)
        dinv = jax.lax.rsqrt(deg.astype(f32))
        vx = jax.lax.dot_general(xb, Vw_ref[...], (((1,), (1,)), ((), ())),
                                 preferred_element_type=f32,
                                 precision=_HIGHEST) + Vb_ref[...]
        agg = jax.lax.dot_general(adj, vx * dinv, (((1,), (0,)), ((), ())),
                                  preferred_element_type=f32,
                                  precision=_HIGHEST) * dinv
        ux = jax.lax.dot_general(xb, Uw_ref[...], (((1,), (1,)), ((), ())),
                                 preferred_element_type=f32,
                                 precision=_HIGHEST) + Ub_ref[...]
        res = agg + ux
        res = (res - bnm_ref[...]) / jnp.sqrt(bnv_ref[...] + 1e-5) \
            * bng_ref[...] + bnb_ref[...]
        x1s.append(jnp.maximum(xb + res, 0.0))

    scale = 1.0 / (HD ** 0.5)
    for s in range(S):
        x1 = x1s[s]
        qkv = jax.lax.dot_general(x1, ipw_ref[...], (((1,), (1,)), ((), ())),
                                  preferred_element_type=f32,
                                  precision=_HIGHEST) + ipb_ref[...]
        heads = []
        for h in range(NHEAD):
            qh = (qkv[:, h * HD:(h + 1) * HD] * scale).astype(bf16)
            kh = qkv[:, D + h * HD:D + (h + 1) * HD].astype(bf16)
            vh = qkv[:, 2 * D + h * HD:2 * D + (h + 1) * HD].astype(bf16)
            sc = jax.lax.dot_general(qh, kh, (((1,), (1,)), ((), ())),
                                     preferred_element_type=f32)
            m = jnp.max(sc, axis=1, keepdims=True)
            e = jnp.exp(sc - m)
            oh = jax.lax.dot_general(e.astype(bf16), vh,
                                     (((1,), (0,)), ((), ())),
                                     preferred_element_type=f32)
            heads.append(oh / _rowsum(e))
        o = jnp.concatenate(heads, axis=1)
        sa = jax.lax.dot_general(o, opw_ref[...], (((1,), (1,)), ((), ())),
                                 preferred_element_type=f32,
                                 precision=_HIGHEST) + opb_ref[...]

        x2 = _layernorm(x1 + sa, ln1g_ref[...], ln1b_ref[...])

        h1 = jnp.maximum(
            jax.lax.dot_general(x2, l1w_ref[...], (((1,), (1,)), ((), ())),
                                preferred_element_type=f32,
                                precision=_HIGHEST) + l1b_ref[...], 0.0)
        ff = jax.lax.dot_general(h1, l2w_ref[...], (((1,), (1,)), ((), ())),
                                 preferred_element_type=f32,
                                 precision=_HIGHEST) + l2b_ref[...]
        out_ref[s] = _layernorm(x2 + ff, ln2g_ref[...], ln2b_ref[...])


def _full(shape):
    return pl.BlockSpec(shape, lambda b: tuple(0 for _ in shape))


def _make_call(interpret=False):
    in_specs = [
        pl.BlockSpec((NSAMP, N, D), lambda b: (b, 0, 0)),
        _full((D, D)), _full((1, D)),
        _full((D, D)), _full((1, D)),
        _full((1, D)), _full((1, D)), _full((1, D)), _full((1, D)),
        _full((3 * D, D)), _full((1, 3 * D)),
        _full((D, D)), _full((1, D)),
        _full((D, D)), _full((1, D)),
        _full((D, D)), _full((1, D)),
        _full((1, D)), _full((1, D)),
        _full((1, D)), _full((1, D)),
    ]
    return pl.pallas_call(
        _gcn_body,
        grid=(8 // NSAMP,),
        in_specs=in_specs,
        out_specs=pl.BlockSpec((NSAMP, N, D), lambda b: (b, 0, 0)),
        out_shape=jax.ShapeDtypeStruct((8, N, D), jnp.float32),
        interpret=interpret,
    )


@jax.jit
def kernel(x, Uw, Ub, Vw, Vb, bn_gamma, bn_beta, bn_mean, bn_var,
           in_proj_w, in_proj_b, out_proj_w, out_proj_b,
           l1_w, l1_b, l2_w, l2_b, ln1_g, ln1_b, ln2_g, ln2_b):
    r = lambda v: v.reshape(1, -1)
    return _make_call()(
        x, Uw, r(Ub), Vw, r(Vb), r(bn_gamma), r(bn_beta), r(bn_mean),
        r(bn_var), in_proj_w, r(in_proj_b), out_proj_w, r(out_proj_b),
        l1_w, r(l1_b), l2_w, r(l2_b), r(ln1_g), r(ln1_b), r(ln2_g),
        r(ln2_b))


# speedup vs baseline: 1.1031x; 1.0065x over previous
import functools

import jax
import jax.numpy as jnp
from jax.experimental import pallas as pl

D = 64
NHEAD = 8
HD = D // NHEAD
KTOP = 32
N = 1024

_HIGHEST = jax.lax.Precision.DEFAULT


def _rowsum(m):
    return jnp.sum(m, axis=1, keepdims=True)


def _layernorm(y, g, b):
    mu = jnp.mean(y, axis=1, keepdims=True)
    c = y - mu
    var = jnp.mean(c * c, axis=1, keepdims=True)
    return c / jnp.sqrt(var + 1e-5) * g + b


NSAMP = 1


def _gcn_body(x_ref, Uw_ref, Ub_ref, Vw_ref, Vb_ref, bng_ref, bnb_ref,
              bnm_ref, bnv_ref, ipw_ref, ipb_ref, opw_ref, opb_ref,
              l1w_ref, l1b_ref, l2w_ref, l2b_ref, ln1g_ref, ln1b_ref,
              ln2g_ref, ln2b_ref, out_ref):
    f32 = jnp.float32
    bf16 = jnp.bfloat16
    S = NSAMP
    xs = [x_ref[i] for i in range(S)]

    sis = []
    for xb in xs:
        nrm = jnp.sqrt(_rowsum(xb * xb))
        sn = xb / jnp.maximum(nrm, 1e-12)
        sis.append(jax.lax.dot_general(sn, sn, (((1,), (1,)), ((), ())),
                                       preferred_element_type=f32,
                                       precision=_HIGHEST))

    keys = []
    for si in sis:
        siz = jnp.where(si == 0.0, 0.0, si)
        bits = jax.lax.bitcast_convert_type(siz, jnp.int32)
        keys.append(bits ^ ((bits >> 31) & jnp.int32(0x7FFFFFFF)))

    lo0 = jnp.full((N, 1), jnp.iinfo(jnp.int32).min, jnp.int32)
    hi0 = jnp.full((N, 1), jnp.iinfo(jnp.int32).max, jnp.int32)
    deg0 = jnp.full((N, 1), N, jnp.int32)

    ones_col = jnp.ones((N // 2, 1), jnp.bfloat16)

    def bs_step(_, carry):
        out = []
        for s in range(S):
            lo, hi, deg = carry[3 * s:3 * s + 3]
            mid = (lo >> 1) + (hi >> 1) + (lo & hi & 1)
            maskA = (keys[s][:, :N // 2] >= mid).astype(jnp.bfloat16)
            cntA = jax.lax.dot_general(maskA, ones_col,
                                       (((1,), (0,)), ((), ())),
                                       preferred_element_type=jnp.float32)
            maskB = (keys[s][:, N // 2:] >= mid).astype(jnp.bfloat16)
            cntB = jax.lax.dot_general(maskB, ones_col,
                                       (((1,), (0,)), ((), ())),
                                       preferred_element_type=jnp.float32)
            cnt = cntA + cntB
            ge = cnt >= KTOP
            out += [jnp.where(ge, mid, lo), jnp.where(ge, hi, mid),
                    jnp.where(ge, cnt, deg)]
        return tuple(out)

    deg0f = jnp.full((N, 1), float(N), jnp.float32)
    fin = jax.lax.fori_loop(0, 32, bs_step, (lo0, hi0, deg0f) * S)

    x1s = []
    for s in range(S):
        lo, deg = fin[3 * s], fin[3 * s + 2]
        xb = xs[s]
        adj = (keys[s] >= lo).astype(f32)
        dinv = jax.lax.rsqrt(deg.astype(f32))
        vx = jax.lax.dot_general(xb, Vw_ref[...], (((1,), (1,)), ((), ())),
                                 preferred_element_type=f32,
                                 precision=_HIGHEST) + Vb_ref[...]
        agg = jax.lax.dot_general(adj, vx * dinv, (((1,), (0,)), ((), ())),
                                  preferred_element_type=f32,
                                  precision=_HIGHEST) * dinv
        ux = jax.lax.dot_general(xb, Uw_ref[...], (((1,), (1,)), ((), ())),
                                 preferred_element_type=f32,
                                 precision=_HIGHEST) + Ub_ref[...]
        res = agg + ux
        res = (res - bnm_ref[...]) / jnp.sqrt(bnv_ref[...] + 1e-5) \
            * bng_ref[...] + bnb_ref[...]
        x1s.append(jnp.maximum(xb + res, 0.0))

    scale = 1.0 / (HD ** 0.5)
    for s in range(S):
        x1 = x1s[s]
        qkv = jax.lax.dot_general(x1, ipw_ref[...], (((1,), (1,)), ((), ())),
                                  preferred_element_type=f32,
                                  precision=_HIGHEST) + ipb_ref[...]
        heads = []
        for h in range(NHEAD):
            qh = (qkv[:, h * HD:(h + 1) * HD] * scale).astype(bf16)
            kh = qkv[:, D + h * HD:D + (h + 1) * HD].astype(bf16)
            vh = qkv[:, 2 * D + h * HD:2 * D + (h + 1) * HD].astype(bf16)
            sc = jax.lax.dot_general(qh, kh, (((1,), (1,)), ((), ())),
                                     preferred_element_type=f32)
            m = jnp.max(sc, axis=1, keepdims=True)
            e = jnp.exp(sc - m)
            oh = jax.lax.dot_general(e.astype(bf16), vh,
                                     (((1,), (0,)), ((), ())),
                                     preferred_element_type=f32)
            heads.append(oh / _rowsum(e))
        o = jnp.concatenate(heads, axis=1)
        sa = jax.lax.dot_general(o, opw_ref[...], (((1,), (1,)), ((), ())),
                                 preferred_element_type=f32,
                                 precision=_HIGHEST) + opb_ref[...]

        x2 = _layernorm(x1 + sa, ln1g_ref[...], ln1b_ref[...])

        h1 = jnp.maximum(
            jax.lax.dot_general(x2, l1w_ref[...], (((1,), (1,)), ((), ())),
                                preferred_element_type=f32,
                                precision=_HIGHEST) + l1b_ref[...], 0.0)
        ff = jax.lax.dot_general(h1, l2w_ref[...], (((1,), (1,)), ((), ())),
                                 preferred_element_type=f32,
                                 precision=_HIGHEST) + l2b_ref[...]
        out_ref[s] = _layernorm(x2 + ff, ln2g_ref[...], ln2b_ref[...])


def _full(shape):
    return pl.BlockSpec(shape, lambda b: tuple(0 for _ in shape))


def _make_call(interpret=False):
    in_specs = [
        pl.BlockSpec((NSAMP, N, D), lambda b: (b, 0, 0)),
        _full((D, D)), _full((1, D)),
        _full((D, D)), _full((1, D)),
        _full((1, D)), _full((1, D)), _full((1, D)), _full((1, D)),
        _full((3 * D, D)), _full((1, 3 * D)),
        _full((D, D)), _full((1, D)),
        _full((D, D)), _full((1, D)),
        _full((D, D)), _full((1, D)),
        _full((1, D)), _full((1, D)),
        _full((1, D)), _full((1, D)),
    ]
    return pl.pallas_call(
        _gcn_body,
        grid=(8 // NSAMP,),
        in_specs=in_specs,
        out_specs=pl.BlockSpec((NSAMP, N, D), lambda b: (b, 0, 0)),
        out_shape=jax.ShapeDtypeStruct((8, N, D), jnp.float32),
        interpret=interpret,
    )


@jax.jit
def kernel(x, Uw, Ub, Vw, Vb, bn_gamma, bn_beta, bn_mean, bn_var,
           in_proj_w, in_proj_b, out_proj_w, out_proj_b,
           l1_w, l1_b, l2_w, l2_b, ln1_g, ln1_b, ln2_g, ln2_b):
    r = lambda v: v.reshape(1, -1)
    return _make_call()(
        x, Uw, r(Ub), Vw, r(Vb), r(bn_gamma), r(bn_beta), r(bn_mean),
        r(bn_var), in_proj_w, r(in_proj_b), out_proj_w, r(out_proj_b),
        l1_w, r(l1_b), l2_w, r(l2_b), r(ln1_g), r(ln1_b), r(ln2_g),
        r(ln2_b))


# speedup vs baseline: 1.1040x; 1.0008x over previous
import jax
import jax.numpy as jnp
from jax.experimental import pallas as pl

D = 64
NHEAD = 8
HD = D // NHEAD
KTOP = 32
N = 1024

_HIGHEST = jax.lax.Precision.DEFAULT


def _rowsum(m):
    return jnp.sum(m, axis=1, keepdims=True)


def _layernorm(y, g, b):
    mu = jnp.mean(y, axis=1, keepdims=True)
    c = y - mu
    var = jnp.mean(c * c, axis=1, keepdims=True)
    return c / jnp.sqrt(var + 1e-5) * g + b


NSAMP = 1


def _gcn_body(x_ref, Uw_ref, Ub_ref, Vw_ref, Vb_ref, bng_ref, bnb_ref,
              bnm_ref, bnv_ref, ipw_ref, ipb_ref, opw_ref, opb_ref,
              l1w_ref, l1b_ref, l2w_ref, l2b_ref, ln1g_ref, ln1b_ref,
              ln2g_ref, ln2b_ref, out_ref):
    f32 = jnp.float32
    bf16 = jnp.bfloat16
    S = NSAMP
    xs = [x_ref[i] for i in range(S)]

    sis = []
    for xb in xs:
        nrm = jnp.sqrt(_rowsum(xb * xb))
        sn = xb / jnp.maximum(nrm, 1e-12)
        sis.append(jax.lax.dot_general(sn, sn, (((1,), (1,)), ((), ())),
                                       preferred_element_type=f32,
                                       precision=_HIGHEST))

    keys = []
    for si in sis:
        siz = jnp.where(si == 0.0, 0.0, si)
        bits = jax.lax.bitcast_convert_type(siz, jnp.int32)
        keys.append(bits ^ ((bits >> 31) & jnp.int32(0x7FFFFFFF)))

    lo0 = jnp.full((N, 1), jnp.iinfo(jnp.int32).min, jnp.int32)
    hi0 = jnp.full((N, 1), jnp.iinfo(jnp.int32).max, jnp.int32)

    ones_col = jnp.ones((N // 2, 1), jnp.bfloat16)

    def bs_step(_, carry):
        out = []
        for s in range(S):
            lo, hi, deg = carry[3 * s:3 * s + 3]
            mid = (lo >> 1) + (hi >> 1) + (lo & hi & 1)
            maskA = (keys[s][:, :N // 2] >= mid).astype(jnp.bfloat16)
            cntA = jax.lax.dot_general(maskA, ones_col,
                                       (((1,), (0,)), ((), ())),
                                       preferred_element_type=jnp.float32)
            maskB = (keys[s][:, N // 2:] >= mid).astype(jnp.bfloat16)
            cntB = jax.lax.dot_general(maskB, ones_col,
                                       (((1,), (0,)), ((), ())),
                                       preferred_element_type=jnp.float32)
            cnt = cntA + cntB
            ge = cnt >= KTOP
            out += [jnp.where(ge, mid, lo), jnp.where(ge, hi, mid),
                    jnp.where(ge, cnt, deg)]
        return tuple(out)

    deg0f = jnp.full((N, 1), float(N), jnp.float32)
    fin = jax.lax.fori_loop(0, 32, bs_step, (lo0, hi0, deg0f) * S)

    x1s = []
    for s in range(S):
        lo, deg = fin[3 * s], fin[3 * s + 2]
        xb = xs[s]
        adj = (keys[s] >= lo).astype(f32)
        dinv = jax.lax.rsqrt(deg.astype(f32))
        vx = jax.lax.dot_general(xb, Vw_ref[...], (((1,), (1,)), ((), ())),
                                 preferred_element_type=f32,
                                 precision=_HIGHEST) + Vb_ref[...]
        agg = jax.lax.dot_general(adj, vx * dinv, (((1,), (0,)), ((), ())),
                                  preferred_element_type=f32,
                                  precision=_HIGHEST) * dinv
        ux = jax.lax.dot_general(xb, Uw_ref[...], (((1,), (1,)), ((), ())),
                                 preferred_element_type=f32,
                                 precision=_HIGHEST) + Ub_ref[...]
        res = agg + ux
        res = (res - bnm_ref[...]) / jnp.sqrt(bnv_ref[...] + 1e-5) \
            * bng_ref[...] + bnb_ref[...]
        x1s.append(jnp.maximum(xb + res, 0.0))

    scale = 1.0 / (HD ** 0.5)
    for s in range(S):
        x1 = x1s[s]
        qkv = jax.lax.dot_general(x1, ipw_ref[...], (((1,), (1,)), ((), ())),
                                  preferred_element_type=f32,
                                  precision=_HIGHEST) + ipb_ref[...]
        heads = []
        for h in range(NHEAD):
            qh = (qkv[:, h * HD:(h + 1) * HD] * scale).astype(bf16)
            kh = qkv[:, D + h * HD:D + (h + 1) * HD].astype(bf16)
            vh = qkv[:, 2 * D + h * HD:2 * D + (h + 1) * HD].astype(bf16)
            sc = jax.lax.dot_general(qh, kh, (((1,), (1,)), ((), ())),
                                     preferred_element_type=f32)
            m = jnp.max(sc, axis=1, keepdims=True)
            e = jnp.exp(sc - m)
            oh = jax.lax.dot_general(e.astype(bf16), vh,
                                     (((1,), (0,)), ((), ())),
                                     preferred_element_type=f32)
            heads.append(oh / _rowsum(e))
        o = jnp.concatenate(heads, axis=1)
        sa = jax.lax.dot_general(o, opw_ref[...], (((1,), (1,)), ((), ())),
                                 preferred_element_type=f32,
                                 precision=_HIGHEST) + opb_ref[...]

        x2 = _layernorm(x1 + sa, ln1g_ref[...], ln1b_ref[...])

        h1 = jnp.maximum(
            jax.lax.dot_general(x2, l1w_ref[...], (((1,), (1,)), ((), ())),
                                preferred_element_type=f32,
                                precision=_HIGHEST) + l1b_ref[...], 0.0)
        ff = jax.lax.dot_general(h1, l2w_ref[...], (((1,), (1,)), ((), ())),
                                 preferred_element_type=f32,
                                 precision=_HIGHEST) + l2b_ref[...]
        out_ref[s] = _layernorm(x2 + ff, ln2g_ref[...], ln2b_ref[...])


def _full(shape):
    return pl.BlockSpec(shape, lambda b: tuple(0 for _ in shape))


def _make_call(interpret=False):
    in_specs = [
        pl.BlockSpec((NSAMP, N, D), lambda b: (b, 0, 0)),
        _full((D, D)), _full((1, D)),
        _full((D, D)), _full((1, D)),
        _full((1, D)), _full((1, D)), _full((1, D)), _full((1, D)),
        _full((3 * D, D)), _full((1, 3 * D)),
        _full((D, D)), _full((1, D)),
        _full((D, D)), _full((1, D)),
        _full((D, D)), _full((1, D)),
        _full((1, D)), _full((1, D)),
        _full((1, D)), _full((1, D)),
    ]
    return pl.pallas_call(
        _gcn_body,
        grid=(8 // NSAMP,),
        in_specs=in_specs,
        out_specs=pl.BlockSpec((NSAMP, N, D), lambda b: (b, 0, 0)),
        out_shape=jax.ShapeDtypeStruct((8, N, D), jnp.float32),
        interpret=interpret,
    )


@jax.jit
def kernel(x, Uw, Ub, Vw, Vb, bn_gamma, bn_beta, bn_mean, bn_var,
           in_proj_w, in_proj_b, out_proj_w, out_proj_b,
           l1_w, l1_b, l2_w, l2_b, ln1_g, ln1_b, ln2_g, ln2_b):
    r = lambda v: v.reshape(1, -1)
    return _make_call()(
        x, Uw, r(Ub), Vw, r(Vb), r(bn_gamma), r(bn_beta), r(bn_mean),
        r(bn_var), in_proj_w, r(in_proj_b), out_proj_w, r(out_proj_b),
        l1_w, r(l1_b), l2_w, r(l2_b), r(ln1_g), r(ln1_b), r(ln2_g),
        r(ln2_b))
